# Initial kernel scaffold; baseline (speedup 1.0000x reference)
import jax
import jax.numpy as jnp
from jax.experimental import pallas as pl


def kernel(x, edge_index, W1, b1, W2, b2, Wm1, bm1, Wm2, bm2):
    raise NotImplementedError("write your pallas kernel here")


# speedup vs baseline: 4.6739x; 4.6739x over previous
import functools

import jax
import jax.numpy as jnp
from jax import lax
from jax.experimental import pallas as pl
from jax.experimental.pallas import tpu as pltpu
from jax.experimental.pallas import tpu_sc as plsc

N = 10000
NP = 10240
E = 160000
NC = 2
NS = 16
K = 80
EPT = E // NS
NCHUNK = EPT // K
RPT = NP // NS
ZR = 32


@functools.lru_cache(maxsize=None)
def _make_seg_sum(nb):
    nbc = nb // NC

    mesh = plsc.VectorSubcoreMesh(
        core_axis_name="c", subcore_axis_name="s",
        num_cores=NC, num_subcores=NS)

    @functools.partial(
        pl.kernel,
        out_type=[jax.ShapeDtypeStruct((NP, 128), jnp.float32)
                  for _ in range(nb)],
        mesh=mesh,
        scratch_types=[
            pltpu.VMEM_SHARED((NP, 128), jnp.float32),
            pltpu.VMEM((NCHUNK, K), jnp.int32),
            pltpu.VMEM((NCHUNK, K), jnp.int32),
            pltpu.VMEM((K, 128), jnp.float32),
            pltpu.VMEM((ZR, 128), jnp.float32),
        ],
    )
    def seg(src_hbm, dst_hbm, *rest):
        feats = rest[:nb]
        outs = rest[nb:2 * nb]
        acc, sidx, didx, rows, zbuf = rest[2 * nb:]
        s = lax.axis_index("s")
        c = lax.axis_index("c")
        row0 = s * RPT

        zv = jnp.zeros((16,), jnp.float32)

        @pl.loop(0, ZR)
        def _(r):
            for j in range(8):
                zbuf[r, pl.ds(j * 16, 16)] = zv

        pltpu.sync_copy(src_hbm.at[s], sidx)
        pltpu.sync_copy(dst_hbm.at[s], didx)

        for core in range(NC):
            for bi in range(nbc):
                blk = core * nbc + bi

                @pl.when(c == core)
                def _(blk=blk):
                    feat = feats[blk]
                    out = outs[blk]
                    for z in range(RPT // ZR):
                        pltpu.sync_copy(
                            zbuf, acc.at[pl.ds(row0 + z * ZR, ZR)])
                    plsc.subcore_barrier()

                    @pl.loop(0, NCHUNK)
                    def _(i):
                        pltpu.sync_copy(feat.at[sidx.at[i]], rows)
                        pltpu.sync_copy(rows, acc.at[didx.at[i]], add=True)

                    plsc.subcore_barrier()
                    pltpu.sync_copy(acc.at[pl.ds(row0, RPT)],
                                    out.at[pl.ds(row0, RPT)])
                    plsc.subcore_barrier()

    return seg


BM = 1280


def _mm1_body(a0, a1, w, b, o0, o1, o2, o3):
    acc = jnp.dot(a0[...], w[0:128, :], preferred_element_type=jnp.float32)
    acc += jnp.dot(a1[...], w[128:256, :], preferred_element_type=jnp.float32)
    h = jnp.maximum(acc + b[...], 0.0)
    o0[...] = h[:, 0:128]
    o1[...] = h[:, 128:256]
    o2[...] = h[:, 256:384]
    o3[...] = h[:, 384:512]


def _mm1(a0, a1, w, b):
    grid = (NP // BM,)
    blk = lambda i: (i, 0)
    cst = lambda i: (0, 0)
    return pl.pallas_call(
        _mm1_body,
        grid=grid,
        in_specs=[
            pl.BlockSpec((BM, 128), blk),
            pl.BlockSpec((BM, 128), blk),
            pl.BlockSpec((256, 512), cst),
            pl.BlockSpec((1, 512), cst),
        ],
        out_specs=[pl.BlockSpec((BM, 128), blk) for _ in range(4)],
        out_shape=[jax.ShapeDtypeStruct((NP, 128), jnp.float32)
                   for _ in range(4)],
    )(a0, a1, w, b)


def _mm2_body(g0, g1, g2, g3, w, b, sums):
    acc = jnp.dot(g0[...], w[0:128, :], preferred_element_type=jnp.float32)
    acc += jnp.dot(g1[...], w[128:256, :], preferred_element_type=jnp.float32)
    acc += jnp.dot(g2[...], w[256:384, :], preferred_element_type=jnp.float32)
    acc += jnp.dot(g3[...], w[384:512, :], preferred_element_type=jnp.float32)
    h = jnp.maximum(acc + b[...], 0.0)
    row = (pl.program_id(0) * BM
           + lax.broadcasted_iota(jnp.int32, h.shape, 0))
    h = jnp.where(row < N, h, 0.0)
    cs = jnp.sum(h, axis=0, keepdims=True)

    @pl.when(pl.program_id(0) == 0)
    def _():
        sums[...] = jnp.zeros_like(sums)

    sums[...] += cs


def _mm2(g0, g1, g2, g3, w, b):
    grid = (NP // BM,)
    blk = lambda i: (i, 0)
    cst = lambda i: (0, 0)
    return pl.pallas_call(
        _mm2_body,
        grid=grid,
        in_specs=[
            pl.BlockSpec((BM, 128), blk),
            pl.BlockSpec((BM, 128), blk),
            pl.BlockSpec((BM, 128), blk),
            pl.BlockSpec((BM, 128), blk),
            pl.BlockSpec((512, 512), cst),
            pl.BlockSpec((1, 512), cst),
        ],
        out_specs=pl.BlockSpec((1, 512), cst),
        out_shape=jax.ShapeDtypeStruct((1, 512), jnp.float32),
    )(g0, g1, g2, g3, w, b)


def _mlp_body(s, wm1, bm1, wm2, bm2, out):
    hg = s[...] * (1.0 / N)
    h = jnp.maximum(
        jnp.dot(hg, wm1[...], preferred_element_type=jnp.float32) + bm1[...],
        0.0)
    out[...] = jnp.maximum(
        jnp.dot(h, wm2[...], preferred_element_type=jnp.float32) + bm2[...],
        0.0)


def _mlp(s, wm1, bm1, wm2, bm2):
    return pl.pallas_call(
        _mlp_body,
        out_shape=jax.ShapeDtypeStruct((1, 256), jnp.float32),
    )(s, wm1, bm1, wm2, bm2)


def kernel(x, edge_index, W1, b1, W2, b2, Wm1, bm1, Wm2, bm2):
    src3d = edge_index[0].reshape(NS, NCHUNK, K)
    dst3d = edge_index[1].reshape(NS, NCHUNK, K)
    x0 = x[:, 0:128]
    x1 = x[:, 128:256]

    a0, a1 = _make_seg_sum(2)(src3d, dst3d, x0, x1)
    h0, h1, h2, h3 = _mm1(a0, a1, W1, b1.reshape(1, 512))
    g0, g1, g2, g3 = _make_seg_sum(4)(src3d, dst3d, h0, h1, h2, h3)
    sums = _mm2(g0, g1, g2, g3, W2, b2.reshape(1, 512))
    return _mlp(sums, Wm1, bm1.reshape(1, 512), Wm2, bm2.reshape(1, 256))
